# baseline (speedup 1.0000x reference)
import functools

import jax
import jax.numpy as jnp
from jax import lax
from jax.experimental import pallas as pl
from jax.experimental.pallas import tpu as pltpu
from jax.experimental.pallas import tpu_sc as plsc

DIM = 128
QDIM = 32
E = 160000
N_NODES = 10000
N_TILES = 16
EP = E // N_TILES
B = 80
NCH = EP // B
ZR = 125


def _sc_kernel_body(dst_hbm, src_hbm, vec_hbm, pv1_hbm, pv2_hbm, ev_hbm,
                    out_hbm, idx_dst, idx_src, vbuf, p1buf, p2buf, evbuf,
                    zbuf, agg):
    c = lax.axis_index("c")
    s = lax.axis_index("s")
    base = s * EP
    rows_per_tile = N_NODES // N_TILES

    def zfill(r, carry):
        zero = jnp.zeros((16,), jnp.float32)
        for k in range(3):
            for h in range(QDIM // 16):
                zbuf[r, k, pl.ds(h * 16, 16)] = zero
        return carry

    lax.fori_loop(0, ZR, zfill, 0)

    for p in range(2):
        d0 = c * (2 * QDIM) + p * QDIM

        for j in range(rows_per_tile // ZR):
            pltpu.sync_copy(
                zbuf, agg.at[pl.ds(s * rows_per_tile + j * ZR, ZR)])
        plsc.subcore_barrier()

        def chunk_body(i, carry):
            e0 = base + i * B
            pltpu.sync_copy(dst_hbm.at[pl.ds(e0, B)], idx_dst)
            pltpu.sync_copy(vec_hbm.at[pl.ds(e0, B), :, pl.ds(d0, QDIM)],
                            vbuf)
            pltpu.sync_copy(pv1_hbm.at[pl.ds(e0, B), pl.ds(d0, QDIM)], p1buf)
            pltpu.sync_copy(pv2_hbm.at[pl.ds(e0, B), pl.ds(d0, QDIM)], p2buf)
            pltpu.sync_copy(ev_hbm.at[pl.ds(e0, B)], evbuf)

            def edge_body(j, ecarry):
                evrow = evbuf[j, :]
                ev0 = evrow[0]
                ev1 = evrow[1]
                ev2 = evrow[2]
                for h in range(QDIM // 16):
                    sl = pl.ds(h * 16, 16)
                    p1 = p1buf[j, sl]
                    p2 = p2buf[j, sl]
                    for k, evk in ((0, ev0), (1, ev1), (2, ev2)):
                        vbuf[j, k, sl] = vbuf[j, k, sl] * p1 + p2 * evk
                return ecarry

            lax.fori_loop(0, B, edge_body, 0)
            pltpu.sync_copy(vbuf, agg.at[idx_dst], add=True)
            return carry

        lax.fori_loop(0, NCH, chunk_body, 0)
        plsc.subcore_barrier()

        def out_body(i, carry):
            e0 = base + i * B
            pltpu.sync_copy(src_hbm.at[pl.ds(e0, B)], idx_src)
            pltpu.sync_copy(agg.at[idx_src], vbuf)
            pltpu.sync_copy(vbuf, out_hbm.at[pl.ds(e0, B), :, pl.ds(d0, QDIM)])
            return carry

        lax.fori_loop(0, NCH, out_body, 0)
        plsc.subcore_barrier()


@jax.jit
def _mpnn_sc(dst, src, vec, pv1, pv2, ev16):
    mesh = plsc.VectorSubcoreMesh(core_axis_name="c", subcore_axis_name="s")
    run = functools.partial(
        pl.kernel,
        mesh=mesh,
        compiler_params=pltpu.CompilerParams(use_tc_tiling_on_sc=False),
        out_type=jax.ShapeDtypeStruct((E, 3, DIM), jnp.float32),
        scratch_types=[
            pltpu.VMEM((B,), jnp.int32),
            pltpu.VMEM((B,), jnp.int32),
            pltpu.VMEM((B, 3, QDIM), jnp.float32),
            pltpu.VMEM((B, QDIM), jnp.float32),
            pltpu.VMEM((B, QDIM), jnp.float32),
            pltpu.VMEM((B, 16), jnp.float32),
            pltpu.VMEM((ZR, 3, QDIM), jnp.float32),
            pltpu.VMEM_SHARED((N_NODES, 3, QDIM), jnp.float32),
        ],
    )(_sc_kernel_body)
    return run(dst, src, vec, pv1, pv2, ev16)


def kernel(edge_index, vec, pos_vec1_list, pos_vec2_list, edge_vec):
    dst = edge_index[:, 1]
    src = edge_index[:, 0]
    pv1 = pos_vec1_list.reshape(E, DIM)
    pv2 = pos_vec2_list.reshape(E, DIM)
    ev16 = jnp.pad(edge_vec.reshape(E, 3), ((0, 0), (0, 13)))
    out = _mpnn_sc(dst, src, vec, pv1, pv2, ev16)
    return out.reshape(1, E, 3, DIM)


# speedup vs baseline: 1.5589x; 1.5589x over previous
import functools

import jax
import jax.numpy as jnp
from jax import lax
from jax.experimental import pallas as pl
from jax.experimental.pallas import tpu as pltpu
from jax.experimental.pallas import tpu_sc as plsc

DIM = 128
QDIM = 32
E = 160000
N_NODES = 10000
N_TILES = 16
EP = E // N_TILES
B = 80
NCH = EP // B
ZR = 125


def _sc_kernel_body(dst_hbm, src_hbm, vec_hbm, pv1_hbm, pv2_hbm, ev_hbm,
                    out_hbm,
                    idx0, idx1, vb0, vb1, p10, p11, p20, p21, ev0b, ev1b,
                    zbuf, agg,
                    s_idx0, s_idx1, s_vec0, s_vec1, s_p10, s_p11,
                    s_p20, s_p21, s_ev0, s_ev1):
    c = lax.axis_index("c")
    s = lax.axis_index("s")
    base = s * EP
    rows_per_tile = N_NODES // N_TILES

    idxb = (idx0, idx1)
    vb = (vb0, vb1)
    p1b = (p10, p11)
    p2b = (p20, p21)
    evb = (ev0b, ev1b)
    sem_idx = (s_idx0, s_idx1)
    sem_vec = (s_vec0, s_vec1)
    sem_p1 = (s_p10, s_p11)
    sem_p2 = (s_p20, s_p21)
    sem_ev = (s_ev0, s_ev1)

    def zfill(r, carry):
        zero = jnp.zeros((16,), jnp.float32)
        for k in range(3):
            for h in range(QDIM // 16):
                zbuf[r, k, pl.ds(h * 16, 16)] = zero
        return carry

    lax.fori_loop(0, ZR, zfill, 0)

    for p in range(2):
        d0 = c * (2 * QDIM) + p * QDIM

        for j in range(rows_per_tile // ZR):
            pltpu.sync_copy(
                zbuf, agg.at[pl.ds(s * rows_per_tile + j * ZR, ZR)])
        plsc.subcore_barrier()

        def p2_descs(cidx, bi):
            e0 = base + cidx * B
            return (
                (dst_hbm.at[pl.ds(e0, B)], idxb[bi], sem_idx[bi]),
                (vec_hbm.at[pl.ds(e0, B), :, pl.ds(d0, QDIM)], vb[bi],
                 sem_vec[bi]),
                (pv1_hbm.at[pl.ds(e0, B), pl.ds(d0, QDIM)], p1b[bi],
                 sem_p1[bi]),
                (pv2_hbm.at[pl.ds(e0, B), pl.ds(d0, QDIM)], p2b[bi],
                 sem_p2[bi]),
                (ev_hbm.at[pl.ds(e0, B)], evb[bi], sem_ev[bi]),
            )

        def p2_start(cidx, bi):
            for src, dstb, sem in p2_descs(cidx, bi):
                pltpu.async_copy(src, dstb, sem)

        def p2_wait(cidx, bi):
            for src, dstb, sem in p2_descs(cidx, bi):
                pltpu.make_async_copy(src, dstb, sem).wait()

        def p2_process(bi):
            vbuf, p1buf, p2buf, evbuf = vb[bi], p1b[bi], p2b[bi], evb[bi]

            def edge_body(j, ecarry):
                evrow = evbuf[j, :]
                e0v = evrow[0]
                e1v = evrow[1]
                e2v = evrow[2]
                for h in range(QDIM // 16):
                    sl = pl.ds(h * 16, 16)
                    q1 = p1buf[j, sl]
                    q2 = p2buf[j, sl]
                    for k, evk in ((0, e0v), (1, e1v), (2, e2v)):
                        vbuf[j, k, sl] = vbuf[j, k, sl] * q1 + q2 * evk
                return ecarry

            lax.fori_loop(0, B, edge_body, 0, unroll=2)
            pltpu.sync_copy(vbuf, agg.at[idxb[bi]], add=True)

        p2_start(0, 0)

        def pair(i2, carry):
            c0 = 2 * i2
            p2_start(c0 + 1, 1)
            p2_wait(c0, 0)
            p2_process(0)
            p2_start(c0 + 2, 0)
            p2_wait(c0 + 1, 1)
            p2_process(1)
            return carry

        lax.fori_loop(0, (NCH - 1) // 2, pair, 0)
        p2_wait(NCH - 1, 0)
        p2_process(0)
        plsc.subcore_barrier()

        def p3_idx_start(cidx, bi):
            e0 = base + cidx * B
            pltpu.async_copy(src_hbm.at[pl.ds(e0, B)], idxb[bi], sem_idx[bi])

        def p3_idx_wait(cidx, bi):
            e0 = base + cidx * B
            pltpu.make_async_copy(
                src_hbm.at[pl.ds(e0, B)], idxb[bi], sem_idx[bi]).wait()

        def p3_out_desc(cidx, bi):
            e0 = base + cidx * B
            return pltpu.make_async_copy(
                vb[bi], out_hbm.at[pl.ds(e0, B), :, pl.ds(d0, QDIM)],
                sem_vec[bi])

        def p3_process(cidx, bi, wait_prev):
            if wait_prev:
                p3_out_desc(cidx - 2, bi).wait()
            pltpu.sync_copy(agg.at[idxb[bi]], vb[bi])
            p3_out_desc(cidx, bi).start()

        p3_idx_start(0, 0)
        p3_idx_start(1, 1)
        p3_idx_wait(0, 0)
        pltpu.sync_copy(agg.at[idxb[0]], vb[0])
        p3_out_desc(0, 0).start()
        p3_idx_start(2, 0)
        p3_idx_wait(1, 1)
        pltpu.sync_copy(agg.at[idxb[1]], vb[1])
        p3_out_desc(1, 1).start()

        def pair3(i2, carry):
            c0 = 2 * i2 + 2
            p3_idx_start(c0 + 1, 1)
            p3_idx_wait(c0, 0)
            p3_process(c0, 0, True)
            p3_idx_start(c0 + 2, 0)
            p3_idx_wait(c0 + 1, 1)
            p3_process(c0 + 1, 1, True)
            return carry

        lax.fori_loop(0, (NCH - 3) // 2, pair3, 0)
        p3_idx_wait(NCH - 1, 0)
        p3_process(NCH - 1, 0, True)
        p3_out_desc(NCH - 2, 1).wait()
        p3_out_desc(NCH - 1, 0).wait()
        plsc.subcore_barrier()


@jax.jit
def _mpnn_sc(dst, src, vec, pv1, pv2, ev16):
    mesh = plsc.VectorSubcoreMesh(core_axis_name="c", subcore_axis_name="s")
    run = functools.partial(
        pl.kernel,
        mesh=mesh,
        compiler_params=pltpu.CompilerParams(use_tc_tiling_on_sc=False),
        out_type=jax.ShapeDtypeStruct((E, 3, DIM), jnp.float32),
        scratch_types=[
            pltpu.VMEM((B,), jnp.int32),
            pltpu.VMEM((B,), jnp.int32),
            pltpu.VMEM((B, 3, QDIM), jnp.float32),
            pltpu.VMEM((B, 3, QDIM), jnp.float32),
            pltpu.VMEM((B, QDIM), jnp.float32),
            pltpu.VMEM((B, QDIM), jnp.float32),
            pltpu.VMEM((B, QDIM), jnp.float32),
            pltpu.VMEM((B, QDIM), jnp.float32),
            pltpu.VMEM((B, 16), jnp.float32),
            pltpu.VMEM((B, 16), jnp.float32),
            pltpu.VMEM((ZR, 3, QDIM), jnp.float32),
            pltpu.VMEM_SHARED((N_NODES, 3, QDIM), jnp.float32),
            pltpu.SemaphoreType.DMA,
            pltpu.SemaphoreType.DMA,
            pltpu.SemaphoreType.DMA,
            pltpu.SemaphoreType.DMA,
            pltpu.SemaphoreType.DMA,
            pltpu.SemaphoreType.DMA,
            pltpu.SemaphoreType.DMA,
            pltpu.SemaphoreType.DMA,
            pltpu.SemaphoreType.DMA,
            pltpu.SemaphoreType.DMA,
        ],
    )(_sc_kernel_body)
    return run(dst, src, vec, pv1, pv2, ev16)


def kernel(edge_index, vec, pos_vec1_list, pos_vec2_list, edge_vec):
    dst = edge_index[:, 1]
    src = edge_index[:, 0]
    pv1 = pos_vec1_list.reshape(E, DIM)
    pv2 = pos_vec2_list.reshape(E, DIM)
    ev16 = jnp.pad(edge_vec.reshape(E, 3), ((0, 0), (0, 13)))
    out = _mpnn_sc(dst, src, vec, pv1, pv2, ev16)
    return out.reshape(1, E, 3, DIM)
